# baseline (speedup 1.0000x reference)
import functools

import jax
import jax.numpy as jnp
from jax.experimental import pallas as pl
from jax.experimental.pallas import tpu as pltpu

_VMEM_LIMIT = 100 * 1024 * 1024


def _mlp_kernel(x_ref, w0_ref, b0_ref, w1_ref, b1_ref, w2_ref, b2_ref,
                of_ref, ob_ref):
    t = x_ref[...].astype(jnp.bfloat16)
    t = jnp.dot(t, w0_ref[...], preferred_element_type=jnp.float32) + b0_ref[...]
    t = jnp.maximum(t, 0.0).astype(jnp.bfloat16)
    t = jnp.dot(t, w1_ref[...], preferred_element_type=jnp.float32) + b1_ref[...]
    t = jnp.maximum(t, 0.0).astype(jnp.bfloat16)
    t = jnp.dot(t, w2_ref[...], preferred_element_type=jnp.float32) + b2_ref[...]
    t = jnp.maximum(t, 0.0)
    of_ref[...] = t
    ob_ref[...] = t.astype(jnp.bfloat16)


def _mlp(x, w0, b0, w1, b1, w2, b2, *, tm):
    n, fin = x.shape
    f0, f1, f2 = w0.shape[1], w1.shape[1], w2.shape[1]
    tm = min(tm, n)
    grid = (n // tm,)
    return pl.pallas_call(
        _mlp_kernel,
        out_shape=(
            jax.ShapeDtypeStruct((n, f2), jnp.float32),
            jax.ShapeDtypeStruct((n, f2), jnp.bfloat16),
        ),
        grid=grid,
        in_specs=[
            pl.BlockSpec((tm, fin), lambda i: (i, 0)),
            pl.BlockSpec((fin, f0), lambda i: (0, 0)),
            pl.BlockSpec((1, f0), lambda i: (0, 0)),
            pl.BlockSpec((f0, f1), lambda i: (0, 0)),
            pl.BlockSpec((1, f1), lambda i: (0, 0)),
            pl.BlockSpec((f1, f2), lambda i: (0, 0)),
            pl.BlockSpec((1, f2), lambda i: (0, 0)),
        ],
        out_specs=(
            pl.BlockSpec((tm, f2), lambda i: (i, 0)),
            pl.BlockSpec((tm, f2), lambda i: (i, 0)),
        ),
        compiler_params=pltpu.CompilerParams(
            dimension_semantics=("parallel",),
            vmem_limit_bytes=_VMEM_LIMIT,
        ),
    )(x, w0, b0, w1, b1, w2, b2)


def _prop_kernel(a_ref, h_ref, x0_ref, of_ref, ob_ref, *, alpha):
    acc = jnp.dot(a_ref[...], h_ref[...], preferred_element_type=jnp.float32)
    o = (1.0 - alpha) * acc + alpha * x0_ref[...]
    of_ref[...] = o
    ob_ref[...] = o.astype(jnp.bfloat16)


def _prop_step(a_hat, h_bf16, x0, *, alpha, tm):
    n, f = x0.shape
    tm = min(tm, n)
    grid = (n // tm,)
    return pl.pallas_call(
        functools.partial(_prop_kernel, alpha=alpha),
        out_shape=(
            jax.ShapeDtypeStruct((n, f), jnp.float32),
            jax.ShapeDtypeStruct((n, f), jnp.bfloat16),
        ),
        grid=grid,
        in_specs=[
            pl.BlockSpec((tm, n), lambda i: (i, 0)),
            pl.BlockSpec((n, f), lambda i: (0, 0)),
            pl.BlockSpec((tm, f), lambda i: (i, 0)),
        ],
        out_specs=(
            pl.BlockSpec((tm, f), lambda i: (i, 0)),
            pl.BlockSpec((tm, f), lambda i: (i, 0)),
        ),
        compiler_params=pltpu.CompilerParams(
            dimension_semantics=("parallel",),
            vmem_limit_bytes=_VMEM_LIMIT,
        ),
    )(a_hat, h_bf16, x0)


def kernel(x, edge_index, w0, w1, w2, b0, b1, b2):
    n = x.shape[0]
    alpha = 0.1
    k_steps = 2

    x0, h_bf16 = _mlp(
        x,
        w0.astype(jnp.bfloat16), b0,
        w1.astype(jnp.bfloat16), b1,
        w2.astype(jnp.bfloat16), b2,
        tm=1024,
    )

    src = edge_index[0]
    dst = edge_index[1]
    deg = jnp.zeros((n,), jnp.float32).at[dst].add(1.0) + 1.0
    dinv = jax.lax.rsqrt(deg)
    loop = jnp.arange(n, dtype=edge_index.dtype)
    rows = jnp.concatenate([dst, loop])
    cols = jnp.concatenate([src, loop])
    vals = (dinv[rows] * dinv[cols]).astype(jnp.bfloat16)
    a_hat = jnp.zeros((n, n), jnp.bfloat16).at[rows, cols].add(vals)

    h = x0
    for _ in range(k_steps):
        h, h_bf16 = _prop_step(a_hat, h_bf16, x0, alpha=alpha, tm=512)
    return h


# speedup vs baseline: 1.9893x; 1.9893x over previous
import functools

import jax
import jax.numpy as jnp
from jax.experimental import pallas as pl
from jax.experimental.pallas import tpu as pltpu

_VMEM_LIMIT = 100 * 1024 * 1024


def _mlp_kernel(x_ref, w0_ref, b0_ref, w1_ref, b1_ref, w2_ref, b2_ref,
                of_ref, ob_ref):
    t = x_ref[...].astype(jnp.bfloat16)
    t = jnp.dot(t, w0_ref[...], preferred_element_type=jnp.float32) + b0_ref[...]
    t = jnp.maximum(t, 0.0).astype(jnp.bfloat16)
    t = jnp.dot(t, w1_ref[...], preferred_element_type=jnp.float32) + b1_ref[...]
    t = jnp.maximum(t, 0.0).astype(jnp.bfloat16)
    t = jnp.dot(t, w2_ref[...], preferred_element_type=jnp.float32) + b2_ref[...]
    t = jnp.maximum(t, 0.0)
    of_ref[...] = t
    ob_ref[...] = t.astype(jnp.bfloat16)


def _mlp(x, w0, b0, w1, b1, w2, b2, *, tm):
    n, fin = x.shape
    f0, f1, f2 = w0.shape[1], w1.shape[1], w2.shape[1]
    tm = min(tm, n)
    grid = (n // tm,)
    return pl.pallas_call(
        _mlp_kernel,
        out_shape=(
            jax.ShapeDtypeStruct((n, f2), jnp.float32),
            jax.ShapeDtypeStruct((n, f2), jnp.bfloat16),
        ),
        grid=grid,
        in_specs=[
            pl.BlockSpec((tm, fin), lambda i: (i, 0)),
            pl.BlockSpec((fin, f0), lambda i: (0, 0)),
            pl.BlockSpec((1, f0), lambda i: (0, 0)),
            pl.BlockSpec((f0, f1), lambda i: (0, 0)),
            pl.BlockSpec((1, f1), lambda i: (0, 0)),
            pl.BlockSpec((f1, f2), lambda i: (0, 0)),
            pl.BlockSpec((1, f2), lambda i: (0, 0)),
        ],
        out_specs=(
            pl.BlockSpec((tm, f2), lambda i: (i, 0)),
            pl.BlockSpec((tm, f2), lambda i: (i, 0)),
        ),
        compiler_params=pltpu.CompilerParams(
            dimension_semantics=("parallel",),
            vmem_limit_bytes=_VMEM_LIMIT,
        ),
    )(x, w0, b0, w1, b1, w2, b2)


def _prop_kernel(c_ref, h_ref, dinv_full_ref, dinv_blk_ref, x0_ref,
                 of_ref, ob_ref, *, alpha):
    g = (h_ref[...].astype(jnp.float32) * dinv_full_ref[...]).astype(jnp.bfloat16)
    acc = jnp.dot(c_ref[...], g, preferred_element_type=jnp.float32)
    o = (1.0 - alpha) * dinv_blk_ref[...] * acc + alpha * x0_ref[...]
    of_ref[...] = o
    ob_ref[...] = o.astype(jnp.bfloat16)


def _prop_step(counts, h_bf16, dinv, x0, *, alpha, tm):
    n, f = x0.shape
    tm = min(tm, n)
    grid = (n // tm,)
    return pl.pallas_call(
        functools.partial(_prop_kernel, alpha=alpha),
        out_shape=(
            jax.ShapeDtypeStruct((n, f), jnp.float32),
            jax.ShapeDtypeStruct((n, f), jnp.bfloat16),
        ),
        grid=grid,
        in_specs=[
            pl.BlockSpec((tm, n), lambda i: (i, 0)),
            pl.BlockSpec((n, f), lambda i: (0, 0)),
            pl.BlockSpec((n, 1), lambda i: (0, 0)),
            pl.BlockSpec((tm, 1), lambda i: (i, 0)),
            pl.BlockSpec((tm, f), lambda i: (i, 0)),
        ],
        out_specs=(
            pl.BlockSpec((tm, f), lambda i: (i, 0)),
            pl.BlockSpec((tm, f), lambda i: (i, 0)),
        ),
        compiler_params=pltpu.CompilerParams(
            dimension_semantics=("parallel",),
            vmem_limit_bytes=_VMEM_LIMIT,
        ),
    )(counts, h_bf16, dinv, dinv, x0)


def kernel(x, edge_index, w0, w1, w2, b0, b1, b2):
    n = x.shape[0]
    alpha = 0.1
    k_steps = 2

    x0, h_bf16 = _mlp(
        x,
        w0.astype(jnp.bfloat16), b0,
        w1.astype(jnp.bfloat16), b1,
        w2.astype(jnp.bfloat16), b2,
        tm=1024,
    )

    src = edge_index[0]
    dst = edge_index[1]
    loop = jnp.arange(n, dtype=edge_index.dtype)
    rows = jnp.concatenate([dst, loop])
    cols = jnp.concatenate([src, loop])
    ones = jnp.ones((rows.shape[0],), jnp.bfloat16)
    counts = jnp.zeros((n, n), jnp.bfloat16).at[rows, cols].add(ones)
    deg = jnp.sum(counts, axis=1, dtype=jnp.float32)
    dinv = jax.lax.rsqrt(jnp.maximum(deg, 1.0))[:, None]

    h = x0
    for _ in range(k_steps):
        h, h_bf16 = _prop_step(counts, h_bf16, dinv, x0, alpha=alpha, tm=512)
    return h


# speedup vs baseline: 3.1744x; 1.5957x over previous
import functools

import jax
import jax.numpy as jnp
from jax.experimental import pallas as pl
from jax.experimental.pallas import tpu as pltpu

_VMEM_LIMIT = 100 * 1024 * 1024


def _mlp_kernel(x_ref, w0_ref, b0_ref, w1_ref, b1_ref, w2_ref, b2_ref, o_ref):
    t = x_ref[...].astype(jnp.bfloat16)
    t = jnp.dot(t, w0_ref[...], preferred_element_type=jnp.float32) + b0_ref[...]
    t = jnp.maximum(t, 0.0).astype(jnp.bfloat16)
    t = jnp.dot(t, w1_ref[...], preferred_element_type=jnp.float32) + b1_ref[...]
    t = jnp.maximum(t, 0.0).astype(jnp.bfloat16)
    t = jnp.dot(t, w2_ref[...], preferred_element_type=jnp.float32) + b2_ref[...]
    o_ref[...] = jnp.maximum(t, 0.0)


def _mlp(x, w0, b0, w1, b1, w2, b2, *, tm):
    n, fin = x.shape
    f0, f1, f2 = w0.shape[1], w1.shape[1], w2.shape[1]
    tm = min(tm, n)
    grid = (n // tm,)
    return pl.pallas_call(
        _mlp_kernel,
        out_shape=jax.ShapeDtypeStruct((n, f2), jnp.float32),
        grid=grid,
        in_specs=[
            pl.BlockSpec((tm, fin), lambda i: (i, 0)),
            pl.BlockSpec((fin, f0), lambda i: (0, 0)),
            pl.BlockSpec((1, f0), lambda i: (0, 0)),
            pl.BlockSpec((f0, f1), lambda i: (0, 0)),
            pl.BlockSpec((1, f1), lambda i: (0, 0)),
            pl.BlockSpec((f1, f2), lambda i: (0, 0)),
            pl.BlockSpec((1, f2), lambda i: (0, 0)),
        ],
        out_specs=pl.BlockSpec((tm, f2), lambda i: (i, 0)),
        compiler_params=pltpu.CompilerParams(
            dimension_semantics=("parallel",),
            vmem_limit_bytes=_VMEM_LIMIT,
        ),
    )(x, w0, b0, w1, b1, w2, b2)


def _prop_kernel(c_ref, h_ref, dinv_full_ref, dinv_blk_ref, x0_ref, o_ref,
                 *, alpha):
    g = h_ref[...] * dinv_full_ref[...]
    acc = jnp.dot(c_ref[...], g, preferred_element_type=jnp.float32)
    o_ref[...] = (1.0 - alpha) * dinv_blk_ref[...] * acc + alpha * x0_ref[...]


def _prop_step(counts, h, dinv, x0, *, alpha, tm):
    n, f = x0.shape
    tm = min(tm, n)
    grid = (n // tm,)
    return pl.pallas_call(
        functools.partial(_prop_kernel, alpha=alpha),
        out_shape=jax.ShapeDtypeStruct((n, f), jnp.float32),
        grid=grid,
        in_specs=[
            pl.BlockSpec((tm, n), lambda i: (i, 0)),
            pl.BlockSpec((n, f), lambda i: (0, 0)),
            pl.BlockSpec((n, 1), lambda i: (0, 0)),
            pl.BlockSpec((tm, 1), lambda i: (i, 0)),
            pl.BlockSpec((tm, f), lambda i: (i, 0)),
        ],
        out_specs=pl.BlockSpec((tm, f), lambda i: (i, 0)),
        compiler_params=pltpu.CompilerParams(
            dimension_semantics=("parallel",),
            vmem_limit_bytes=_VMEM_LIMIT,
        ),
    )(counts, h, dinv, dinv, x0)


def kernel(x, edge_index, w0, w1, w2, b0, b1, b2):
    n = x.shape[0]
    alpha = 0.1
    k_steps = 2

    x0 = _mlp(
        x,
        w0.astype(jnp.bfloat16), b0,
        w1.astype(jnp.bfloat16), b1,
        w2.astype(jnp.bfloat16), b2,
        tm=1024,
    )

    src = edge_index[0]
    dst = edge_index[1]
    loop = jnp.arange(n, dtype=edge_index.dtype)
    rows = jnp.concatenate([dst, loop])
    cols = jnp.concatenate([src, loop])
    ones = jnp.ones((rows.shape[0],), jnp.float32)
    counts = jnp.zeros((n, n), jnp.float32).at[rows, cols].add(ones)

    ones_e = jnp.ones((dst.shape[0],), jnp.float32)
    deg_acc = jnp.zeros((n, 128), jnp.float32).at[dst, dst % 128].add(ones_e)
    deg = jnp.sum(deg_acc, axis=1) + 1.0
    dinv = jax.lax.rsqrt(deg)[:, None]

    h = x0
    for _ in range(k_steps):
        h = _prop_step(counts, h, dinv, x0, alpha=alpha, tm=512)
    return h


# speedup vs baseline: 3.5829x; 1.1287x over previous
import functools

import jax
import jax.numpy as jnp
from jax.experimental import pallas as pl
from jax.experimental.pallas import tpu as pltpu

_VMEM_LIMIT = 100 * 1024 * 1024


def _mlp_kernel(x_ref, w0_ref, b0_ref, w1_ref, b1_ref, w2_ref, b2_ref, o_ref):
    t = x_ref[...].astype(jnp.bfloat16)
    t = jnp.dot(t, w0_ref[...], preferred_element_type=jnp.float32) + b0_ref[...]
    t = jnp.maximum(t, 0.0).astype(jnp.bfloat16)
    t = jnp.dot(t, w1_ref[...], preferred_element_type=jnp.float32) + b1_ref[...]
    t = jnp.maximum(t, 0.0).astype(jnp.bfloat16)
    t = jnp.dot(t, w2_ref[...], preferred_element_type=jnp.float32) + b2_ref[...]
    o_ref[...] = jnp.maximum(t, 0.0)


def _mlp(x, w0, b0, w1, b1, w2, b2, *, tm):
    n, fin = x.shape
    f0, f1, f2 = w0.shape[1], w1.shape[1], w2.shape[1]
    tm = min(tm, n)
    grid = (n // tm,)
    return pl.pallas_call(
        _mlp_kernel,
        out_shape=jax.ShapeDtypeStruct((n, f2), jnp.float32),
        grid=grid,
        in_specs=[
            pl.BlockSpec((tm, fin), lambda i: (i, 0)),
            pl.BlockSpec((fin, f0), lambda i: (0, 0)),
            pl.BlockSpec((1, f0), lambda i: (0, 0)),
            pl.BlockSpec((f0, f1), lambda i: (0, 0)),
            pl.BlockSpec((1, f1), lambda i: (0, 0)),
            pl.BlockSpec((f1, f2), lambda i: (0, 0)),
            pl.BlockSpec((1, f2), lambda i: (0, 0)),
        ],
        out_specs=pl.BlockSpec((tm, f2), lambda i: (i, 0)),
        compiler_params=pltpu.CompilerParams(
            dimension_semantics=("parallel",),
            vmem_limit_bytes=_VMEM_LIMIT,
        ),
    )(x, w0, b0, w1, b1, w2, b2)


def _prop_kernel(c_ref, h_ref, dinv_full_ref, dinv_blk_ref, x0_ref, o_ref,
                 *, alpha, tm):
    g = h_ref[...] * dinv_full_ref[...]
    acc = jnp.dot(c_ref[...], g, preferred_element_type=jnp.float32)
    i = pl.program_id(0)
    g_blk = h_ref[pl.ds(i * tm, tm), :] * dinv_blk_ref[...]
    o_ref[...] = ((1.0 - alpha) * dinv_blk_ref[...] * (acc + g_blk)
                  + alpha * x0_ref[...])


def _prop_step(counts, h, dinv, x0, *, alpha, tm):
    n, f = x0.shape
    tm = min(tm, n)
    grid = (n // tm,)
    return pl.pallas_call(
        functools.partial(_prop_kernel, alpha=alpha, tm=tm),
        out_shape=jax.ShapeDtypeStruct((n, f), jnp.float32),
        grid=grid,
        in_specs=[
            pl.BlockSpec((tm, n), lambda i: (i, 0)),
            pl.BlockSpec((n, f), lambda i: (0, 0)),
            pl.BlockSpec((n, 1), lambda i: (0, 0)),
            pl.BlockSpec((tm, 1), lambda i: (i, 0)),
            pl.BlockSpec((tm, f), lambda i: (i, 0)),
        ],
        out_specs=pl.BlockSpec((tm, f), lambda i: (i, 0)),
        compiler_params=pltpu.CompilerParams(
            dimension_semantics=("parallel",),
            vmem_limit_bytes=_VMEM_LIMIT,
        ),
    )(counts, h, dinv, dinv, x0)


def kernel(x, edge_index, w0, w1, w2, b0, b1, b2):
    n = x.shape[0]
    alpha = 0.1
    k_steps = 2

    x0 = _mlp(
        x,
        w0.astype(jnp.bfloat16), b0,
        w1.astype(jnp.bfloat16), b1,
        w2.astype(jnp.bfloat16), b2,
        tm=1024,
    )

    src = edge_index[0]
    dst = edge_index[1]
    ones = jnp.ones((dst.shape[0],), jnp.float32)
    counts = jnp.zeros((n, n), jnp.float32).at[dst, src].add(ones)
    deg = jnp.sum(counts, axis=1) + 1.0
    dinv = jax.lax.rsqrt(deg)[:, None]

    h = x0
    for _ in range(k_steps):
        h = _prop_step(counts, h, dinv, x0, alpha=alpha, tm=512)
    return h


# speedup vs baseline: 4.7154x; 1.3161x over previous
import functools

import jax
import jax.numpy as jnp
from jax.experimental import pallas as pl
from jax.experimental.pallas import tpu as pltpu

_VMEM_LIMIT = 100 * 1024 * 1024


def _mlp_kernel(x_ref, w0_ref, b0_ref, w1_ref, b1_ref, w2_ref, b2_ref, o_ref):
    t = x_ref[...].astype(jnp.bfloat16)
    t = jnp.dot(t, w0_ref[...], preferred_element_type=jnp.float32) + b0_ref[...]
    t = jnp.maximum(t, 0.0).astype(jnp.bfloat16)
    t = jnp.dot(t, w1_ref[...], preferred_element_type=jnp.float32) + b1_ref[...]
    t = jnp.maximum(t, 0.0).astype(jnp.bfloat16)
    t = jnp.dot(t, w2_ref[...], preferred_element_type=jnp.float32) + b2_ref[...]
    o_ref[...] = jnp.maximum(t, 0.0)


def _mlp(x, w0, b0, w1, b1, w2, b2, *, tm):
    n, fin = x.shape
    f0, f1, f2 = w0.shape[1], w1.shape[1], w2.shape[1]
    tm = min(tm, n)
    grid = (n // tm,)
    return pl.pallas_call(
        _mlp_kernel,
        out_shape=jax.ShapeDtypeStruct((n, f2), jnp.float32),
        grid=grid,
        in_specs=[
            pl.BlockSpec((tm, fin), lambda i: (i, 0)),
            pl.BlockSpec((fin, f0), lambda i: (0, 0)),
            pl.BlockSpec((1, f0), lambda i: (0, 0)),
            pl.BlockSpec((f0, f1), lambda i: (0, 0)),
            pl.BlockSpec((1, f1), lambda i: (0, 0)),
            pl.BlockSpec((f1, f2), lambda i: (0, 0)),
            pl.BlockSpec((1, f2), lambda i: (0, 0)),
        ],
        out_specs=pl.BlockSpec((tm, f2), lambda i: (i, 0)),
        compiler_params=pltpu.CompilerParams(
            dimension_semantics=("parallel",),
            vmem_limit_bytes=_VMEM_LIMIT,
        ),
    )(x, w0, b0, w1, b1, w2, b2)


def _prop_kernel(c_ref, h_ref, dinv_full_ref, dinv_blk_ref, x0_ref, o_ref,
                 *, alpha, tm, nsub):
    g = h_ref[...] * dinv_full_ref[...]
    acc = jnp.dot(c_ref[0:tm, :], g[0:128, :],
                  preferred_element_type=jnp.float32)
    for k in range(1, nsub):
        acc += jnp.dot(c_ref[k * tm:(k + 1) * tm, :],
                       g[k * 128:(k + 1) * 128, :],
                       preferred_element_type=jnp.float32)
    i = pl.program_id(0)
    g_blk = h_ref[pl.ds(i * tm, tm), :] * dinv_blk_ref[...]
    o_ref[...] = ((1.0 - alpha) * dinv_blk_ref[...] * (acc + g_blk)
                  + alpha * x0_ref[...])


def _prop_step(counts, h, dinv, x0, *, alpha, tm):
    n, f = x0.shape
    tm = min(tm, n)
    nsub = n // 128
    grid = (n // tm,)
    return pl.pallas_call(
        functools.partial(_prop_kernel, alpha=alpha, tm=tm, nsub=nsub),
        out_shape=jax.ShapeDtypeStruct((n, f), jnp.float32),
        grid=grid,
        in_specs=[
            pl.BlockSpec((tm * nsub, 128), lambda i: (i, 0)),
            pl.BlockSpec((n, f), lambda i: (0, 0)),
            pl.BlockSpec((n, 1), lambda i: (0, 0)),
            pl.BlockSpec((tm, 1), lambda i: (i, 0)),
            pl.BlockSpec((tm, f), lambda i: (i, 0)),
        ],
        out_specs=pl.BlockSpec((tm, f), lambda i: (i, 0)),
        compiler_params=pltpu.CompilerParams(
            dimension_semantics=("parallel",),
            vmem_limit_bytes=_VMEM_LIMIT,
        ),
    )(counts, h, dinv, dinv, x0)


def kernel(x, edge_index, w0, w1, w2, b0, b1, b2):
    n = x.shape[0]
    alpha = 0.1
    k_steps = 2

    x0 = _mlp(
        x,
        w0.astype(jnp.bfloat16), b0,
        w1.astype(jnp.bfloat16), b1,
        w2.astype(jnp.bfloat16), b2,
        tm=1024,
    )

    src = edge_index[0]
    dst = edge_index[1]
    tm = min(512, n)
    nsub = n // 128
    row = (dst // tm) * (tm * nsub) + (src // 128) * tm + (dst % tm)
    pos = row * 128 + (src % 128)
    ones = jnp.ones((dst.shape[0],), jnp.float32)
    flat = jnp.zeros((n * n,), jnp.float32).at[pos].add(ones)
    counts_slabs = flat.reshape(n * nsub, 128)
    deg = (counts_slabs.reshape(n // tm, nsub, tm, 128)
           .sum(axis=(1, 3)).reshape(n)) + 1.0
    dinv = jax.lax.rsqrt(deg)[:, None]

    h = x0
    for _ in range(k_steps):
        h = _prop_step(counts_slabs, h, dinv, x0, alpha=alpha, tm=tm)
    return h
